# baseline (speedup 1.0000x reference)
import functools

import jax
import jax.numpy as jnp
from jax import lax
from jax.experimental import pallas as pl
from jax.experimental.pallas import tpu as pltpu
from jax.experimental.pallas import tpu_sc as plsc

N = 10000
E = 320000
D = 128
L = 16
NC = 2
NS = 16
CH = 128
B = 8
NG = 20
NCH = NG * B
EPT = NCH * CH
E_PAD = EPT * NS
NROW_BLK = 128
N_PAD = 10240
BLK_PER_SC = N_PAD // NROW_BLK // NS
ROWS_OUT = N_PAD // NS


def _spmm_body(x_hbm, src_hbm, dst_hbm, w_hbm, out_hbm,
               srcb, dstb, wb, rows0, rows1, accum, gsem0, gsem1, ssem):
    c = lax.axis_index("c")
    s = lax.axis_index("s")
    rows = (rows0, rows1)
    gsem = (gsem0, gsem1)

    def zero_rows(i, carry):
        z = jnp.zeros((L,), jnp.float32)
        for j in range(D // L):
            rows0[i, pl.ds(j * L, L)] = z
        return carry

    lax.fori_loop(0, CH, zero_rows, 0)

    def zero_accum(k, carry):
        blk = (s * BLK_PER_SC + k) * NROW_BLK
        pltpu.sync_copy(rows0, accum.at[pl.ds(blk, NROW_BLK)])
        return carry

    lax.fori_loop(0, BLK_PER_SC, zero_accum, 0)
    plsc.subcore_barrier()

    def weight_mul(e, rows_v):
        def grp_body(gg, carry):
            wv = wb[e, pl.ds(gg * L, L)]
            for k in range(L):
                we = wv[k]
                r = gg * L + k
                for j in range(D // L):
                    rows_v[r, pl.ds(j * L, L)] = rows_v[r, pl.ds(j * L, L)] * we
            return carry

        lax.fori_loop(0, CH // L, grp_body, 0)

    def group_body(g, carry):
        pltpu.sync_copy(src_hbm.at[c, s, g], srcb)
        pltpu.sync_copy(dst_hbm.at[c, s, g], dstb)
        pltpu.sync_copy(w_hbm.at[c, s, g], wb)
        pltpu.async_copy(x_hbm.at[srcb.at[0]], rows0, gsem0)
        for e in range(B):
            cur_rows, nxt_rows = rows[e % 2], rows[(e + 1) % 2]
            cur_gsem, nxt_gsem = gsem[e % 2], gsem[(e + 1) % 2]
            pltpu.make_async_copy(
                x_hbm.at[srcb.at[e]], cur_rows, cur_gsem).wait()
            if e < B - 1:
                pltpu.async_copy(x_hbm.at[srcb.at[e + 1]], nxt_rows, nxt_gsem)
            weight_mul(e, cur_rows)
            pltpu.async_copy(
                cur_rows, accum.at[dstb.at[e]], ssem, add=True).wait()
        return carry

    lax.fori_loop(0, NG, group_body, 0)
    plsc.subcore_barrier()

    pltpu.sync_copy(accum.at[pl.ds(s * ROWS_OUT, ROWS_OUT)],
                    out_hbm.at[c, pl.ds(s * ROWS_OUT, ROWS_OUT)])


@jax.jit
def _sc_spmm(x, src, dst, w):
    mesh = plsc.VectorSubcoreMesh(core_axis_name="c", subcore_axis_name="s")
    f = functools.partial(
        pl.kernel,
        out_type=jax.ShapeDtypeStruct((NC, N_PAD, D), jnp.float32),
        mesh=mesh,
        scratch_types=[
            pltpu.VMEM((B, CH), jnp.int32),
            pltpu.VMEM((B, CH), jnp.int32),
            pltpu.VMEM((B, CH), jnp.float32),
            pltpu.VMEM((CH, D), jnp.float32),
            pltpu.VMEM((CH, D), jnp.float32),
            pltpu.VMEM_SHARED((N_PAD, D), jnp.float32),
            pltpu.SemaphoreType.DMA,
            pltpu.SemaphoreType.DMA,
            pltpu.SemaphoreType.DMA,
        ],
    )(_spmm_body)
    return f(x, src, dst, w)


def kernel(x, edge_index1, edge_weight1, edge_index2, edge_weight2):
    pad = E_PAD - E
    src = jnp.pad(jnp.stack([edge_index1[1], edge_index2[1]]),
                  ((0, 0), (0, pad))).reshape(NC, NS, NG, B, CH)
    dst = jnp.pad(jnp.stack([edge_index1[0], edge_index2[0]]),
                  ((0, 0), (0, pad))).reshape(NC, NS, NG, B, CH)
    w = jnp.pad(jnp.stack([edge_weight1, edge_weight2]),
                ((0, 0), (0, pad))).reshape(NC, NS, NG, B, CH)
    out = _sc_spmm(x, src, dst, w)
    return jnp.concatenate([out[0, :N], out[1, :N]], axis=1)
